# initial kernel scaffold (unmeasured)
import jax
import jax.numpy as jnp
from jax import lax
from jax.experimental import pallas as pl
from jax.experimental.pallas import tpu as pltpu

N_DEV = 32
M_BLK = 128


def kernel(x, w_mat, scale_x, scale_w):
    m_global, k_per = x.shape
    k_global, n = w_mat.shape

    def body(x_ref, w_ref, sx_ref, sw_ref, out_ref, comm_ref,
             send_sems, recv_sems):
        me = lax.axis_index("i")

        comm_ref[:, pl.ds(me * M_BLK, M_BLK)] = x_ref[pl.ds(me * M_BLK, M_BLK), :]

        for off in range(1, N_DEV):
            j = lax.rem(me + off, N_DEV)
            rdma = pltpu.make_async_remote_copy(
                src_ref=x_ref.at[pl.ds(j * M_BLK, M_BLK), :],
                dst_ref=comm_ref.at[:, pl.ds(me * M_BLK, M_BLK)],
                send_sem=send_sems.at[off],
                recv_sem=recv_sems.at[off],
                device_id=(j,),
                device_id_type=pl.DeviceIdType.MESH,
            )
            rdma.start()

        for off in range(1, N_DEV):
            src = lax.rem(me - off + N_DEV, N_DEV)
            recv = pltpu.make_async_remote_copy(
                src_ref=x_ref.at[pl.ds(0, M_BLK), :],
                dst_ref=comm_ref.at[:, pl.ds(src * M_BLK, M_BLK)],
                send_sem=send_sems.at[off],
                recv_sem=recv_sems.at[off],
                device_id=(me,),
                device_id_type=pl.DeviceIdType.MESH,
            )
            recv.wait_recv()

        acc = jnp.dot(comm_ref[:, :], w_ref[:, :],
                      preferred_element_type=jnp.int32)
        s = sx_ref[0] * sw_ref[0]
        out_ref[:, :] = jnp.maximum(acc.astype(jnp.float32) * s, 0.0)

        for off in range(1, N_DEV):
            snd = pltpu.make_async_remote_copy(
                src_ref=x_ref.at[pl.ds(0, M_BLK), :],
                dst_ref=comm_ref.at[:, pl.ds(0, M_BLK)],
                send_sem=send_sems.at[off],
                recv_sem=recv_sems.at[off],
                device_id=(me,),
                device_id_type=pl.DeviceIdType.MESH,
            )
            snd.wait_send()

    return pl.pallas_call(
        body,
        out_shape=jax.ShapeDtypeStruct((M_BLK, n), jnp.float32),
        in_specs=[
            pl.BlockSpec(memory_space=pltpu.VMEM),
            pl.BlockSpec(memory_space=pltpu.VMEM),
            pl.BlockSpec(memory_space=pltpu.SMEM),
            pl.BlockSpec(memory_space=pltpu.SMEM),
        ],
        out_specs=pl.BlockSpec(memory_space=pltpu.VMEM),
        scratch_shapes=[
            pltpu.VMEM((M_BLK, k_global), jnp.int8),
            pltpu.SemaphoreType.DMA((N_DEV,)),
            pltpu.SemaphoreType.DMA((N_DEV,)),
        ],
        compiler_params=pltpu.CompilerParams(collective_id=0),
    )(x, w_mat, scale_x, scale_w)


# baseline (device time: 47384 ns/iter reference)
import jax
import jax.numpy as jnp
from jax import lax
from jax.experimental import pallas as pl
from jax.experimental.pallas import tpu as pltpu

N_DEV = 32
M_BLK = 128


def kernel(x, w_mat, scale_x, scale_w):
    m_global, k_per = x.shape
    k_global, n = w_mat.shape

    def body(x_ref, w_ref, sx_ref, sw_ref, out_ref, comm_ref,
             send_sems, recv_sems):
        me = lax.axis_index("i")

        comm_ref[:, pl.ds(me * M_BLK, M_BLK)] = x_ref[pl.ds(me * M_BLK, M_BLK), :]

        for off in range(1, N_DEV):
            j = lax.rem(me + off, N_DEV)
            rdma = pltpu.make_async_remote_copy(
                src_ref=x_ref.at[pl.ds(j * M_BLK, M_BLK), :],
                dst_ref=comm_ref.at[:, pl.ds(me * M_BLK, M_BLK)],
                send_sem=send_sems.at[off],
                recv_sem=recv_sems.at[off],
                device_id=(j,),
                device_id_type=pl.DeviceIdType.MESH,
            )
            rdma.start()

        for off in range(1, N_DEV):
            src = lax.rem(me - off + N_DEV, N_DEV)
            recv = pltpu.make_async_remote_copy(
                src_ref=x_ref.at[pl.ds(0, M_BLK), :],
                dst_ref=comm_ref.at[:, pl.ds(src * M_BLK, M_BLK)],
                send_sem=send_sems.at[off],
                recv_sem=recv_sems.at[off],
                device_id=(me,),
                device_id_type=pl.DeviceIdType.MESH,
            )
            recv.wait_recv()

        acc = jnp.dot(comm_ref[:, :], w_ref[:, :],
                      preferred_element_type=jnp.int32)
        s = sx_ref[0] * sw_ref[0]
        out_ref[:, :] = jnp.maximum(acc.astype(jnp.float32) * s, 0.0)

        for off in range(1, N_DEV):
            snd = pltpu.make_async_remote_copy(
                src_ref=x_ref.at[pl.ds(0, M_BLK), :],
                dst_ref=comm_ref.at[:, pl.ds(0, M_BLK)],
                send_sem=send_sems.at[off],
                recv_sem=recv_sems.at[off],
                device_id=(me,),
                device_id_type=pl.DeviceIdType.MESH,
            )
            snd.wait_send()

    return pl.pallas_call(
        body,
        out_shape=jax.ShapeDtypeStruct((M_BLK, n), jnp.float32),
        in_specs=[
            pl.BlockSpec(memory_space=pltpu.VMEM),
            pl.BlockSpec(memory_space=pltpu.VMEM),
            pl.BlockSpec(memory_space=pltpu.SMEM),
            pl.BlockSpec(memory_space=pltpu.SMEM),
        ],
        out_specs=pl.BlockSpec(memory_space=pltpu.VMEM),
        scratch_shapes=[
            pltpu.VMEM((M_BLK, k_global), jnp.int8),
            pltpu.SemaphoreType.DMA((N_DEV,)),
            pltpu.SemaphoreType.DMA((N_DEV,)),
        ],
        compiler_params=pltpu.CompilerParams(
            vmem_limit_bytes=64 * 1024 * 1024,
        ),
    )(x, w_mat, scale_x, scale_w)
